# baseline (speedup 1.0000x reference)
import functools
import math as _math

import jax
import jax.numpy as jnp
from jax import lax
from jax.experimental import pallas as pl
from jax.experimental.pallas import tpu as pltpu
from jax.experimental.pallas import tpu_sc as plsc

_SIZE = 100000
_N = 1024
_SMOOTHING = 0.1
_CONF = 1.0 - _SMOOTHING
_S = _SMOOTHING / (_SIZE - 2)
_C1 = _CONF * _math.log(_CONF) + _SMOOTHING * _math.log(_S)

_BC = 2560
_NBC = 31
_CTC = _NBC * _BC
_CEND = 99840
_TAILB = _CEND // _BC


def _tc_body(t_ref, x_ref, o_ref):
    k = pl.program_id(0)
    xb = x_ref[...]
    m = (t_ref[...] != 0).astype(jnp.float32)

    edge = jnp.logical_or(k == 0, k == _NBC)

    @pl.when(edge)
    def _():
        cs = jnp.where(k == _NBC, _TAILB * _BC, 0)
        col = lax.broadcasted_iota(jnp.int32, xb.shape, 1) + cs
        valid = jnp.logical_and(col >= 1, col < _SIZE)
        rows = jnp.sum(jnp.where(valid, xb, 0.0), axis=1, keepdims=True)
        part = -_S * (rows * m)
        part = part + jnp.where(k == 0, _C1 * m, 0.0)

        @pl.when(k == 0)
        def _():
            o_ref[...] = part

        @pl.when(k != 0)
        def _():
            o_ref[...] += part

    @pl.when(jnp.logical_not(edge))
    def _():
        rows = jnp.sum(xb, axis=1, keepdims=True)
        o_ref[...] += -_S * (rows * m)


def _tc_reduce(x, t2d):
    return pl.pallas_call(
        _tc_body,
        grid=(_NBC + 1,),
        in_specs=[
            pl.BlockSpec((_N, 1), lambda k: (0, 0)),
            pl.BlockSpec((_N, _BC), lambda k: (0, jnp.where(k == _NBC, _TAILB, k))),
        ],
        out_specs=pl.BlockSpec((_N, 1), lambda k: (0, 0)),
        out_shape=jax.ShapeDtypeStruct((_N, 1), jnp.float32),
        compiler_params=pltpu.CompilerParams(
            dimension_semantics=("arbitrary",),
        ),
    )(t2d, x)


_L = 16
_CW = 512
_NCH = (_CEND - _CTC) // _CW


def _make_sc_kernel(nw):
    bpw = _N // nw
    mesh = plsc.VectorSubcoreMesh(core_axis_name="c", subcore_axis_name="s")
    info = plsc.get_sparse_core_info()
    nc = info.num_cores

    @functools.partial(
        pl.kernel,
        mesh=mesh,
        out_type=jax.ShapeDtypeStruct((nw * 2 * _L,), jnp.float32),
        scratch_types=[
            pltpu.VMEM((bpw,), jnp.int32),
            pltpu.VMEM((bpw, 8, 128), jnp.float32),
            pltpu.VMEM((bpw, _CW), jnp.float32),
            pltpu.VMEM((bpw, _CW), jnp.float32),
            pltpu.VMEM((_L,), jnp.float32),
            pltpu.SemaphoreType.DMA,
            pltpu.SemaphoreType.DMA,
            pltpu.SemaphoreType.DMA,
        ],
    )
    def sc_kernel(
        tgt_hbm, x_hbm, out_hbm, tgt_v, win_v, buf_a, buf_b, stage_v, gsem, sem_a, sem_b
    ):
        wid = lax.axis_index("s") * nc + lax.axis_index("c")
        base = wid * bpw
        pltpu.sync_copy(tgt_hbm.at[pl.ds(base, bpw)], tgt_v)
        lane = lax.iota(jnp.int32, _L)

        ts = []
        gcopies = []
        for j in range(bpw):
            tj = tgt_v[pl.ds((j // _L) * _L, _L)][j % _L]
            al = pl.multiple_of(lax.bitwise_and(tj, jnp.int32(~127)), 128)
            ts.append((tj, al))
            gcopies.append(
                pltpu.async_copy(
                    x_hbm.at[pl.ds(base + (j & ~7), 8), pl.ds(al, 128)],
                    win_v.at[j],
                    gsem,
                )
            )
        flags = [
            jnp.where(tj != 0, jnp.float32(1.0), jnp.float32(0.0)) for tj, _ in ts
        ]

        def chunk_src(c):
            cs = pl.multiple_of(_CTC + c * _CW, 128)
            return x_hbm.at[pl.ds(base, bpw), pl.ds(cs, _CW)]

        def reduce_buf(buf):
            tot = jnp.zeros((_L,), jnp.float32)
            for r in range(bpw):
                rs = buf[r, pl.ds(0, _L)]
                for v in range(1, _CW // _L):
                    rs = rs + buf[r, pl.ds(v * _L, _L)]
                tot = tot + rs * flags[r]
            return tot

        pltpu.async_copy(chunk_src(0), buf_a, sem_a)

        def pair_body(i, acc):
            c0 = 2 * i
            pltpu.async_copy(chunk_src(c0 + 1), buf_b, sem_b)
            pltpu.make_async_copy(chunk_src(0), buf_a, sem_a).wait()
            acc = acc + reduce_buf(buf_a)

            @pl.when(c0 + 2 < _NCH)
            def _():
                pltpu.async_copy(chunk_src(c0 + 2), buf_a, sem_a)

            pltpu.make_async_copy(chunk_src(0), buf_b, sem_b).wait()
            acc = acc + reduce_buf(buf_b)
            return acc

        dacc = lax.fori_loop(
            0, _NCH // 2, pair_body, jnp.zeros((_L,), jnp.float32)
        )

        for cp in gcopies:
            cp.wait()
        gacc = jnp.zeros((_L,), jnp.float32)
        for j in range(bpw):
            tj, al = ts[j]
            rem = tj - al
            hi = pl.multiple_of(lax.shift_right_logical(rem, 4) * _L, _L)
            lo = lax.bitwise_and(rem, 15)
            w = win_v[j, j & 7, pl.ds(hi, _L)]
            lo = jnp.where(tj != 0, lo, jnp.int32(-1))
            gacc = gacc + jnp.where(lane == lo, w, 0.0)

        stage_v[...] = gacc
        pltpu.sync_copy(stage_v, out_hbm.at[pl.ds(wid * 2 * _L, _L)])
        stage_v[...] = dacc
        pltpu.sync_copy(stage_v, out_hbm.at[pl.ds(wid * 2 * _L + _L, _L)])

    return sc_kernel


def kernel(x, target):
    t32 = target.astype(jnp.int32)
    t2d = t32.reshape(_N, 1)
    tc_parts = _tc_reduce(x, t2d)

    info = plsc.get_sparse_core_info()
    nw = info.num_cores * info.num_subcores
    sc_parts = _make_sc_kernel(nw)(t32, x).reshape(nw, 2, _L)

    g = jnp.sum(sc_parts[:, 0, :])
    dn = jnp.sum(sc_parts[:, 1, :])
    return jnp.sum(tc_parts) - jnp.float32(_CONF - _S) * g - jnp.float32(_S) * dn


# speedup vs baseline: 1.0007x; 1.0007x over previous
import functools
import math as _math

import jax
import jax.numpy as jnp
from jax import lax
from jax.experimental import pallas as pl
from jax.experimental.pallas import tpu as pltpu
from jax.experimental.pallas import tpu_sc as plsc

_SIZE = 100000
_N = 1024
_SMOOTHING = 0.1
_CONF = 1.0 - _SMOOTHING
_S = _SMOOTHING / (_SIZE - 2)
_C1 = _CONF * _math.log(_CONF) + _SMOOTHING * _math.log(_S)

_BC = 2560
_NBC = 31
_CTC = _NBC * _BC
_CEND = 99840
_TAILB = _CEND // _BC


def _tc_body(t_ref, x_ref, o_ref):
    k = pl.program_id(0)
    xb = x_ref[...]
    m = (t_ref[...] != 0).astype(jnp.float32)

    edge = jnp.logical_or(k == 0, k == _NBC)

    @pl.when(edge)
    def _():
        cs = jnp.where(k == _NBC, _TAILB * _BC, 0)
        col = lax.broadcasted_iota(jnp.int32, xb.shape, 1) + cs
        valid = jnp.logical_and(col >= 1, col < _SIZE)
        rows = jnp.sum(jnp.where(valid, xb, 0.0), axis=1, keepdims=True)
        part = -_S * (rows * m)
        part = part + jnp.where(k == 0, _C1 * m, 0.0)

        @pl.when(k == 0)
        def _():
            o_ref[...] = part

        @pl.when(k != 0)
        def _():
            o_ref[...] += part

    @pl.when(jnp.logical_not(edge))
    def _():
        rows = jnp.sum(xb, axis=1, keepdims=True)
        o_ref[...] += -_S * (rows * m)


def _tc_reduce(x, t2d):
    return pl.pallas_call(
        _tc_body,
        grid=(_NBC + 1,),
        in_specs=[
            pl.BlockSpec((_N, 1), lambda k: (0, 0)),
            pl.BlockSpec((_N, _BC), lambda k: (0, jnp.where(k == _NBC, _TAILB, k))),
        ],
        out_specs=pl.BlockSpec((_N, 1), lambda k: (0, 0)),
        out_shape=jax.ShapeDtypeStruct((_N, 1), jnp.float32),
        compiler_params=pltpu.CompilerParams(
            dimension_semantics=("arbitrary",),
        ),
    )(t2d, x)


_L = 16
_CW = 512
_NCH = (_CEND - _CTC) // _CW


def _make_sc_kernel(nw):
    bpw = _N // nw
    mesh = plsc.VectorSubcoreMesh(core_axis_name="c", subcore_axis_name="s")
    info = plsc.get_sparse_core_info()
    nc = info.num_cores

    @functools.partial(
        pl.kernel,
        mesh=mesh,
        out_type=jax.ShapeDtypeStruct((nw * 2 * _L,), jnp.float32),
        scratch_types=[
            pltpu.VMEM((bpw,), jnp.int32),
            pltpu.VMEM((bpw, 8, 128), jnp.float32),
            pltpu.VMEM((bpw, _CW), jnp.float32),
            pltpu.VMEM((bpw, _CW), jnp.float32),
            pltpu.VMEM((_L,), jnp.float32),
            pltpu.SemaphoreType.DMA,
            pltpu.SemaphoreType.DMA,
            pltpu.SemaphoreType.DMA,
        ],
        compiler_params=pltpu.CompilerParams(use_tc_tiling_on_sc=True),
    )
    def sc_kernel(
        tgt_hbm, x_hbm, out_hbm, tgt_v, win_v, buf_a, buf_b, stage_v, gsem, sem_a, sem_b
    ):
        wid = lax.axis_index("s") * nc + lax.axis_index("c")
        base = wid * bpw
        pltpu.sync_copy(tgt_hbm.at[pl.ds(base, bpw)], tgt_v)
        lane = lax.iota(jnp.int32, _L)

        ts = []
        gcopies = []
        for j in range(bpw):
            tj = tgt_v[pl.ds((j // _L) * _L, _L)][j % _L]
            al = pl.multiple_of(lax.bitwise_and(tj, jnp.int32(~127)), 128)
            ts.append((tj, al))
            gcopies.append(
                pltpu.async_copy(
                    x_hbm.at[pl.ds(base + (j & ~7), 8), pl.ds(al, 128)],
                    win_v.at[j],
                    gsem,
                )
            )
        flags = [
            jnp.where(tj != 0, jnp.float32(1.0), jnp.float32(0.0)) for tj, _ in ts
        ]

        def chunk_src(c):
            cs = pl.multiple_of(_CTC + c * _CW, 128)
            return x_hbm.at[pl.ds(base, bpw), pl.ds(cs, _CW)]

        def reduce_buf(buf):
            tot = jnp.zeros((_L,), jnp.float32)
            for r in range(bpw):
                rs = buf[r, pl.ds(0, _L)]
                for v in range(1, _CW // _L):
                    rs = rs + buf[r, pl.ds(v * _L, _L)]
                tot = tot + rs * flags[r]
            return tot

        pltpu.async_copy(chunk_src(0), buf_a, sem_a)

        def pair_body(i, acc):
            c0 = 2 * i
            pltpu.async_copy(chunk_src(c0 + 1), buf_b, sem_b)
            pltpu.make_async_copy(chunk_src(0), buf_a, sem_a).wait()
            acc = acc + reduce_buf(buf_a)

            @pl.when(c0 + 2 < _NCH)
            def _():
                pltpu.async_copy(chunk_src(c0 + 2), buf_a, sem_a)

            pltpu.make_async_copy(chunk_src(0), buf_b, sem_b).wait()
            acc = acc + reduce_buf(buf_b)
            return acc

        dacc = lax.fori_loop(
            0, _NCH // 2, pair_body, jnp.zeros((_L,), jnp.float32)
        )

        for cp in gcopies:
            cp.wait()
        gacc = jnp.zeros((_L,), jnp.float32)
        for j in range(bpw):
            tj, al = ts[j]
            rem = tj - al
            hi = pl.multiple_of(lax.shift_right_logical(rem, 4) * _L, _L)
            lo = lax.bitwise_and(rem, 15)
            w = win_v[j, j & 7, pl.ds(hi, _L)]
            lo = jnp.where(tj != 0, lo, jnp.int32(-1))
            gacc = gacc + jnp.where(lane == lo, w, 0.0)

        stage_v[...] = gacc
        pltpu.sync_copy(stage_v, out_hbm.at[pl.ds(wid * 2 * _L, _L)])
        stage_v[...] = dacc
        pltpu.sync_copy(stage_v, out_hbm.at[pl.ds(wid * 2 * _L + _L, _L)])

    return sc_kernel


def kernel(x, target):
    t32 = target.astype(jnp.int32)
    t2d = t32.reshape(_N, 1)
    tc_parts = _tc_reduce(x, t2d)

    info = plsc.get_sparse_core_info()
    nw = info.num_cores * info.num_subcores
    sc_parts = _make_sc_kernel(nw)(t32, x).reshape(nw, 2, _L)

    g = jnp.sum(sc_parts[:, 0, :])
    dn = jnp.sum(sc_parts[:, 1, :])
    return jnp.sum(tc_parts) - jnp.float32(_CONF - _S) * g - jnp.float32(_S) * dn


# speedup vs baseline: 3.8893x; 3.8868x over previous
import functools
import math as _math

import jax
import jax.numpy as jnp
from jax import lax
from jax.experimental import pallas as pl
from jax.experimental.pallas import tpu as pltpu
from jax.experimental.pallas import tpu_sc as plsc

_SIZE = 100000
_N = 1024
_SMOOTHING = 0.1
_CONF = 1.0 - _SMOOTHING
_S = _SMOOTHING / (_SIZE - 2)
_C1 = _CONF * _math.log(_CONF) + _SMOOTHING * _math.log(_S)

_BR = 1376
_NBT = 63
_RTC = _BR * _NBT
_QR = (_SIZE - _RTC) // 4


def _tc_body(t_ref, x_ref, o_ref, acc_ref):
    k = pl.program_id(0)
    xb = x_ref[...]

    @pl.when(k == 0)
    def _():
        row = lax.broadcasted_iota(jnp.int32, xb.shape, 0)
        acc_ref[...] = jnp.sum(
            jnp.where(row == 0, 0.0, xb), axis=0, keepdims=True
        )

    @pl.when(k != 0)
    def _():
        acc_ref[...] += jnp.sum(xb, axis=0, keepdims=True)

    @pl.when(k == _NBT - 1)
    def _():
        m = (t_ref[...] != 0).astype(jnp.float32)
        total = _C1 * jnp.sum(m) - _S * jnp.sum(m * acc_ref[...])
        o_ref[...] = jnp.reshape(total, (1, 1))


def _tc_reduce(xt, t2d):
    return pl.pallas_call(
        _tc_body,
        grid=(_NBT,),
        in_specs=[
            pl.BlockSpec((1, _N), lambda k: (0, 0)),
            pl.BlockSpec((_BR, _N), lambda k: (k, 0)),
        ],
        out_specs=pl.BlockSpec((1, 1), lambda k: (0, 0)),
        out_shape=jax.ShapeDtypeStruct((1, 1), jnp.float32),
        scratch_shapes=[pltpu.VMEM((1, _N), jnp.float32)],
        compiler_params=pltpu.CompilerParams(
            dimension_semantics=("arbitrary",),
        ),
    )(t2d, xt)


_L = 16
_CR = 64
_NCH = _QR // _CR


def _make_sc_kernel(nw):
    bpw = _N // nw
    mesh = plsc.VectorSubcoreMesh(core_axis_name="c", subcore_axis_name="s")
    info = plsc.get_sparse_core_info()
    nc = info.num_cores

    @functools.partial(
        pl.kernel,
        mesh=mesh,
        out_type=jax.ShapeDtypeStruct((nw * 2 * _L,), jnp.float32),
        scratch_types=[
            pltpu.VMEM((bpw,), jnp.int32),
            pltpu.VMEM((128,), jnp.int32),
            pltpu.VMEM((bpw, 8, 128), jnp.float32),
            pltpu.VMEM((_CR, 128), jnp.float32),
            pltpu.VMEM((_CR, 128), jnp.float32),
            pltpu.VMEM((_L,), jnp.float32),
            pltpu.SemaphoreType.DMA,
            pltpu.SemaphoreType.DMA,
            pltpu.SemaphoreType.DMA,
        ],
        compiler_params=pltpu.CompilerParams(use_tc_tiling_on_sc=True),
    )
    def sc_kernel(
        tgt_hbm,
        xt_hbm,
        out_hbm,
        tg_v,
        tgd_v,
        win_v,
        buf_a,
        buf_b,
        stage_v,
        gsem,
        sem_a,
        sem_b,
    ):
        wid = lax.axis_index("s") * nc + lax.axis_index("c")
        base = wid * bpw
        cb = pl.multiple_of((wid & 7) * 128, 128)
        q = lax.shift_right_logical(wid, 3)
        pltpu.sync_copy(tgt_hbm.at[pl.ds(base, bpw)], tg_v)
        pltpu.sync_copy(tgt_hbm.at[pl.ds(cb, 128)], tgd_v)
        lane = lax.iota(jnp.int32, _L)

        colw = pl.multiple_of(lax.bitwise_and(base, ~127), 128)
        coff = base - colw

        ts = []
        gcopies = []
        for j in range(bpw):
            tj = tg_v[pl.ds((j // _L) * _L, _L)][j % _L]
            t_al = pl.multiple_of(lax.bitwise_and(tj, jnp.int32(~7)), 8)
            ts.append((tj, t_al))
            gcopies.append(
                pltpu.async_copy(
                    xt_hbm.at[pl.ds(t_al, 8), pl.ds(colw, 128)],
                    win_v.at[j],
                    gsem,
                )
            )

        def chunk_src(c):
            rs = pl.multiple_of(_RTC + q * _QR + c * _CR, 8)
            return xt_hbm.at[pl.ds(rs, _CR), pl.ds(cb, 128)]

        def reduce_buf(buf, accs):
            out = []
            for g in range(8):
                sg = accs[g]
                for r in range(_CR):
                    sg = sg + buf[r, pl.ds(g * _L, _L)]
                out.append(sg)
            return tuple(out)

        pltpu.async_copy(chunk_src(0), buf_a, sem_a)
        zero = jnp.zeros((_L,), jnp.float32)

        def pair_body(i, accs):
            c0 = 2 * i
            pltpu.async_copy(chunk_src(c0 + 1), buf_b, sem_b)
            pltpu.make_async_copy(chunk_src(0), buf_a, sem_a).wait()
            accs = reduce_buf(buf_a, accs)

            @pl.when(c0 + 2 < _NCH)
            def _():
                pltpu.async_copy(chunk_src(c0 + 2), buf_a, sem_a)

            pltpu.make_async_copy(chunk_src(0), buf_b, sem_b).wait()
            accs = reduce_buf(buf_b, accs)
            return accs

        accs = lax.fori_loop(0, _NCH // 2, pair_body, (zero,) * 8)

        dacc = zero
        for g in range(8):
            tflag = jnp.where(
                tgd_v[pl.ds(g * _L, _L)] == 0, jnp.float32(0.0), jnp.float32(1.0)
            )
            dacc = dacc + accs[g] * tflag

        for cp in gcopies:
            cp.wait()
        gacc = zero
        for j in range(bpw):
            tj, t_al = ts[j]
            rj = tj - t_al
            grp = pl.multiple_of(coff + (j // _L) * _L, _L)
            lo = j % _L
            for r in range(8):
                w = win_v[j, r, pl.ds(grp, _L)]
                sel = jnp.where(
                    jnp.logical_and(tj != 0, rj == r), jnp.int32(lo), jnp.int32(-1)
                )
                gacc = gacc + jnp.where(lane == sel, w, 0.0)

        stage_v[...] = gacc
        pltpu.sync_copy(stage_v, out_hbm.at[pl.ds(wid * 2 * _L, _L)])
        stage_v[...] = dacc
        pltpu.sync_copy(stage_v, out_hbm.at[pl.ds(wid * 2 * _L + _L, _L)])

    return sc_kernel


def kernel(x, target):
    t32 = target.astype(jnp.int32)
    xt = x.T
    tc_out = _tc_reduce(xt, t32.reshape(1, _N))

    info = plsc.get_sparse_core_info()
    nw = info.num_cores * info.num_subcores
    sc_parts = _make_sc_kernel(nw)(t32, xt).reshape(nw, 2, _L)

    g = jnp.sum(sc_parts[:, 0, :])
    dn = jnp.sum(sc_parts[:, 1, :])
    return tc_out[0, 0] - jnp.float32(_CONF - _S) * g - jnp.float32(_S) * dn


# speedup vs baseline: 3.9753x; 1.0221x over previous
import functools
import math as _math

import jax
import jax.numpy as jnp
from jax import lax
from jax.experimental import pallas as pl
from jax.experimental.pallas import tpu as pltpu
from jax.experimental.pallas import tpu_sc as plsc

_SIZE = 100000
_N = 1024
_SMOOTHING = 0.1
_CONF = 1.0 - _SMOOTHING
_S = _SMOOTHING / (_SIZE - 2)
_C1 = _CONF * _math.log(_CONF) + _SMOOTHING * _math.log(_S)

_BR = 2016
_NBT = 43
_RTC = _BR * _NBT
_QR = (_SIZE - _RTC) // 4


def _tc_body(t_ref, x_ref, o_ref, acc_ref):
    k = pl.program_id(0)
    xb = x_ref[...]

    @pl.when(k == 0)
    def _():
        row = lax.broadcasted_iota(jnp.int32, xb.shape, 0)
        acc_ref[...] = jnp.sum(
            jnp.where(row == 0, 0.0, xb), axis=0, keepdims=True
        )

    @pl.when(k != 0)
    def _():
        acc_ref[...] += jnp.sum(xb, axis=0, keepdims=True)

    @pl.when(k == _NBT - 1)
    def _():
        m = (t_ref[...] != 0).astype(jnp.float32)
        total = _C1 * jnp.sum(m) - _S * jnp.sum(m * acc_ref[...])
        o_ref[...] = jnp.reshape(total, (1, 1))


def _tc_reduce(xt, t2d):
    return pl.pallas_call(
        _tc_body,
        grid=(_NBT,),
        in_specs=[
            pl.BlockSpec((1, _N), lambda k: (0, 0)),
            pl.BlockSpec((_BR, _N), lambda k: (k, 0)),
        ],
        out_specs=pl.BlockSpec((1, 1), lambda k: (0, 0)),
        out_shape=jax.ShapeDtypeStruct((1, 1), jnp.float32),
        scratch_shapes=[pltpu.VMEM((1, _N), jnp.float32)],
        compiler_params=pltpu.CompilerParams(
            dimension_semantics=("arbitrary",),
        ),
    )(t2d, xt)


_L = 16
_CR = 64
_NCH = _QR // _CR


def _make_sc_kernel(nw):
    bpw = _N // nw
    mesh = plsc.VectorSubcoreMesh(core_axis_name="c", subcore_axis_name="s")
    info = plsc.get_sparse_core_info()
    nc = info.num_cores

    @functools.partial(
        pl.kernel,
        mesh=mesh,
        out_type=jax.ShapeDtypeStruct((nw * 2 * _L,), jnp.float32),
        scratch_types=[
            pltpu.VMEM((bpw,), jnp.int32),
            pltpu.VMEM((128,), jnp.int32),
            pltpu.VMEM((bpw, 8, 128), jnp.float32),
            pltpu.VMEM((_CR, 128), jnp.float32),
            pltpu.VMEM((_CR, 128), jnp.float32),
            pltpu.VMEM((_L,), jnp.float32),
            pltpu.SemaphoreType.DMA,
            pltpu.SemaphoreType.DMA,
            pltpu.SemaphoreType.DMA,
        ],
        compiler_params=pltpu.CompilerParams(use_tc_tiling_on_sc=True),
    )
    def sc_kernel(
        tgt_hbm,
        xt_hbm,
        out_hbm,
        tg_v,
        tgd_v,
        win_v,
        buf_a,
        buf_b,
        stage_v,
        gsem,
        sem_a,
        sem_b,
    ):
        wid = lax.axis_index("s") * nc + lax.axis_index("c")
        base = wid * bpw
        cb = pl.multiple_of((wid & 7) * 128, 128)
        q = lax.shift_right_logical(wid, 3)
        pltpu.sync_copy(tgt_hbm.at[pl.ds(base, bpw)], tg_v)
        pltpu.sync_copy(tgt_hbm.at[pl.ds(cb, 128)], tgd_v)
        lane = lax.iota(jnp.int32, _L)

        colw = pl.multiple_of(lax.bitwise_and(base, ~127), 128)
        coff = base - colw

        ts = []
        gcopies = []
        for j in range(bpw):
            tj = tg_v[pl.ds((j // _L) * _L, _L)][j % _L]
            t_al = pl.multiple_of(lax.bitwise_and(tj, jnp.int32(~7)), 8)
            ts.append((tj, t_al))
            gcopies.append(
                pltpu.async_copy(
                    xt_hbm.at[pl.ds(t_al, 8), pl.ds(colw, 128)],
                    win_v.at[j],
                    gsem,
                )
            )

        def chunk_src(c):
            rs = pl.multiple_of(_RTC + q * _QR + c * _CR, 8)
            return xt_hbm.at[pl.ds(rs, _CR), pl.ds(cb, 128)]

        def reduce_buf(buf, accs):
            out = []
            for g in range(8):
                sg = accs[g]
                for r in range(_CR):
                    sg = sg + buf[r, pl.ds(g * _L, _L)]
                out.append(sg)
            return tuple(out)

        pltpu.async_copy(chunk_src(0), buf_a, sem_a)
        zero = jnp.zeros((_L,), jnp.float32)

        def pair_body(i, accs):
            c0 = 2 * i
            pltpu.async_copy(chunk_src(c0 + 1), buf_b, sem_b)
            pltpu.make_async_copy(chunk_src(0), buf_a, sem_a).wait()
            accs = reduce_buf(buf_a, accs)

            @pl.when(c0 + 2 < _NCH)
            def _():
                pltpu.async_copy(chunk_src(c0 + 2), buf_a, sem_a)

            pltpu.make_async_copy(chunk_src(0), buf_b, sem_b).wait()
            accs = reduce_buf(buf_b, accs)
            return accs

        accs = lax.fori_loop(0, _NCH // 2, pair_body, (zero,) * 8)

        dacc = zero
        for g in range(8):
            tflag = jnp.where(
                tgd_v[pl.ds(g * _L, _L)] == 0, jnp.float32(0.0), jnp.float32(1.0)
            )
            dacc = dacc + accs[g] * tflag

        for cp in gcopies:
            cp.wait()
        gacc = zero
        for j in range(bpw):
            tj, t_al = ts[j]
            rj = tj - t_al
            grp = pl.multiple_of(coff + (j // _L) * _L, _L)
            lo = j % _L
            for r in range(8):
                w = win_v[j, r, pl.ds(grp, _L)]
                sel = jnp.where(
                    jnp.logical_and(tj != 0, rj == r), jnp.int32(lo), jnp.int32(-1)
                )
                gacc = gacc + jnp.where(lane == sel, w, 0.0)

        stage_v[...] = gacc
        pltpu.sync_copy(stage_v, out_hbm.at[pl.ds(wid * 2 * _L, _L)])
        stage_v[...] = dacc
        pltpu.sync_copy(stage_v, out_hbm.at[pl.ds(wid * 2 * _L + _L, _L)])

    return sc_kernel


def kernel(x, target):
    t32 = target.astype(jnp.int32)
    xt = x.T
    tc_out = _tc_reduce(xt, t32.reshape(1, _N))

    info = plsc.get_sparse_core_info()
    nw = info.num_cores * info.num_subcores
    sc_parts = _make_sc_kernel(nw)(t32, xt).reshape(nw, 2, _L)

    g = jnp.sum(sc_parts[:, 0, :])
    dn = jnp.sum(sc_parts[:, 1, :])
    return tc_out[0, 0] - jnp.float32(_CONF - _S) * g - jnp.float32(_S) * dn
